# baseline (speedup 1.0000x reference)
import jax
import jax.numpy as jnp
from jax.experimental import pallas as pl
from jax.experimental.pallas import tpu as pltpu

B, S, D = 1, 2048, 768
E, TOPK = 8, 2
I = 341
BT = 2048


def _moe_body(x_ref, gw_ref, sw_ref, sd_ref, ew_ref, ed_ref, out_ref,
              wsel_ref, xb_ref):
    e = pl.program_id(1)

    def ffn(xb, w_gu, w_d):
        gu = jnp.dot(xb, w_gu, preferred_element_type=jnp.float32)
        g = gu[:, :I]
        u = gu[:, I:]
        h = (g * jax.nn.sigmoid(g)) * u
        return jnp.dot(h.astype(jnp.bfloat16), w_d,
                       preferred_element_type=jnp.float32)

    @pl.when(e == 0)
    def _router_and_shared():
        xt = x_ref[...]
        logits = jnp.dot(xt, gw_ref[...], preferred_element_type=jnp.float32)
        lane = jax.lax.broadcasted_iota(jnp.int32, (BT, E), 1)
        m = jnp.max(logits, axis=1, keepdims=True)
        ex = jnp.exp(logits - m)
        probs = ex / jnp.sum(ex, axis=1, keepdims=True)
        p1 = jnp.max(probs, axis=1, keepdims=True)
        sel1 = jnp.min(jnp.where(probs == p1, lane, E), axis=1, keepdims=True)
        probs2 = jnp.where(lane == sel1, -1.0, probs)
        p2 = jnp.max(probs2, axis=1, keepdims=True)
        sel2 = jnp.min(jnp.where(probs2 == p2, lane, E), axis=1, keepdims=True)
        wsum = p1 + p2 + 1e-8
        wsel_ref[...] = (jnp.where(lane == sel1, p1, 0.0)
                         + jnp.where(lane == sel2, p2, 0.0)) / wsum
        xb = xt.astype(jnp.bfloat16)
        xb_ref[...] = xb
        out_ref[...] = ffn(xb, sw_ref[...], sd_ref[...])

    @pl.when(e > 0)
    def _expert():
        ye = ffn(xb_ref[...], ew_ref[0], ed_ref[0])
        lane8 = jax.lax.broadcasted_iota(jnp.int32, (E, 1), 0)
        onehot = (lane8 == e - 1).astype(jnp.float32)
        w_col = jnp.dot(wsel_ref[...], onehot,
                        preferred_element_type=jnp.float32)
        out_ref[...] += w_col * ye


@jax.jit
def _moe(x, gw, sw, sd, ew, ed):
    grid = (S // BT, E + 1)
    return pl.pallas_call(
        _moe_body,
        grid=grid,
        in_specs=[
            pl.BlockSpec((BT, D), lambda i, e: (i, 0)),
            pl.BlockSpec((D, E), lambda i, e: (0, 0)),
            pl.BlockSpec((D, 2 * I), lambda i, e: (0, 0)),
            pl.BlockSpec((I, D), lambda i, e: (0, 0)),
            pl.BlockSpec((1, D, 2 * I),
                         lambda i, e: (jnp.maximum(e - 1, 0), 0, 0)),
            pl.BlockSpec((1, I, D),
                         lambda i, e: (jnp.maximum(e - 1, 0), 0, 0)),
        ],
        out_specs=pl.BlockSpec((BT, D), lambda i, e: (i, 0)),
        out_shape=jax.ShapeDtypeStruct((S, D), jnp.float32),
        scratch_shapes=[
            pltpu.VMEM((BT, E), jnp.float32),
            pltpu.VMEM((BT, D), jnp.bfloat16),
        ],
    )(x, gw, sw, sd, ew, ed)


def kernel(hidden_states, shared_gate_up_w, shared_down_w, expert_gate_up,
           expert_down, gate_weight):
    b, s, d = hidden_states.shape
    x = hidden_states.reshape(s, d)
    gw = gate_weight.T
    sw = shared_gate_up_w.T.astype(jnp.bfloat16)
    sd = shared_down_w.T.astype(jnp.bfloat16)
    ew = expert_gate_up.astype(jnp.bfloat16)
    ed = expert_down.astype(jnp.bfloat16)
    out = _moe(x, gw, sw, sd, ew, ed)
    return out.reshape(b, s, d)
